# Initial kernel scaffold; baseline (speedup 1.0000x reference)
import functools

import jax
import jax.numpy as jnp
from jax import lax
from jax.experimental import pallas as pl
from jax.experimental.pallas import tpu as pltpu
from jax.experimental.pallas import tpu_sc as plsc

NC = 2
NS = 16
NW = NC * NS
K = 128

_MESH = plsc.VectorSubcoreMesh(core_axis_name="c", subcore_axis_name="s")


def _make_scatter_add(n_pad, feat, nb):
    rpt = n_pad // NS

    @functools.partial(
        pl.kernel,
        out_type=jax.ShapeDtypeStruct((NC, n_pad, feat), jnp.float32),
        mesh=_MESH,
        scratch_types=[
            pltpu.VMEM_SHARED((n_pad, feat), jnp.float32),
            pltpu.VMEM((nb, K), jnp.int32),
            pltpu.VMEM((nb, K), jnp.int32),
            pltpu.VMEM((K, feat), jnp.float32),
            pltpu.SemaphoreType.DMA,
        ],
    )
    def scatter_kernel(table_hbm, src_hbm, dst_hbm, zeros_hbm, out_hbm,
                       acc, src_v, dst_v, rows_v, sem):
        c = lax.axis_index("c")
        s = lax.axis_index("s")
        wid = s * NC + c
        r0 = s * rpt
        pltpu.sync_copy(zeros_hbm.at[pl.ds(r0, rpt)], acc.at[pl.ds(r0, rpt)])
        pltpu.sync_copy(src_hbm.at[pl.ds(wid * nb, nb)], src_v)
        pltpu.sync_copy(dst_hbm.at[pl.ds(wid * nb, nb)], dst_v)
        plsc.subcore_barrier()

        def body(j, carry):
            pltpu.async_copy(table_hbm.at[src_v.at[j]], rows_v, sem).wait()
            pltpu.sync_copy(rows_v, acc.at[dst_v.at[j]], add=True)
            return carry

        lax.fori_loop(0, nb, body, 0)
        plsc.subcore_barrier()
        pltpu.sync_copy(acc.at[pl.ds(r0, rpt)], out_hbm.at[c, pl.ds(r0, rpt)])

    return scatter_kernel


def _make_degree(n_pad, nb):
    rpt = n_pad // NS

    @functools.partial(
        pl.kernel,
        out_type=jax.ShapeDtypeStruct((NC, n_pad, 16), jnp.float32),
        mesh=_MESH,
        scratch_types=[
            pltpu.VMEM_SHARED((n_pad, 16), jnp.float32),
            pltpu.VMEM((nb, K), jnp.int32),
            pltpu.VMEM((K, 16), jnp.float32),
        ],
    )
    def deg_kernel(dst_hbm, ones_hbm, zeros_hbm, out_hbm, acc, dst_v, ones_v):
        c = lax.axis_index("c")
        s = lax.axis_index("s")
        wid = s * NC + c
        r0 = s * rpt
        pltpu.sync_copy(zeros_hbm.at[pl.ds(r0, rpt)], acc.at[pl.ds(r0, rpt)])
        pltpu.sync_copy(dst_hbm.at[pl.ds(wid * nb, nb)], dst_v)
        pltpu.sync_copy(ones_hbm, ones_v)
        plsc.subcore_barrier()

        def body(j, carry):
            pltpu.sync_copy(ones_v, acc.at[dst_v.at[j]], add=True)
            return carry

        lax.fori_loop(0, nb, body, 0)
        plsc.subcore_barrier()
        pltpu.sync_copy(acc.at[pl.ds(r0, rpt)], out_hbm.at[c, pl.ds(r0, rpt)])

    return deg_kernel


def _tc_scale(dega, degb, x, block):
    n, f = x.shape

    def body(da_ref, db_ref, x_ref, dinv_ref, xs_ref):
        d = da_ref[...] + db_ref[...] + 1.0
        dinv = 1.0 / jnp.sqrt(jnp.maximum(d[:, 0:1], 1.0))
        dinv_ref[...] = dinv
        xs_ref[...] = x_ref[...] * dinv

    return pl.pallas_call(
        body,
        grid=(n // block,),
        in_specs=[
            pl.BlockSpec((block, 16), lambda i: (i, 0)),
            pl.BlockSpec((block, 16), lambda i: (i, 0)),
            pl.BlockSpec((block, f), lambda i: (i, 0)),
        ],
        out_specs=[
            pl.BlockSpec((block, 1), lambda i: (i, 0)),
            pl.BlockSpec((block, f), lambda i: (i, 0)),
        ],
        out_shape=[
            jax.ShapeDtypeStruct((n, 1), jnp.float32),
            jax.ShapeDtypeStruct((n, f), jnp.float32),
        ],
    )(dega, degb, x)


def _tc_layer1(o1a, o1b, xs, dinv, W1, b1r, W2p, block):
    n, f = xs.shape
    k1 = W1.shape[1]
    f2 = W2p.shape[1]

    def body(oa_ref, ob_ref, xs_ref, dinv_ref, w1_ref, b1_ref, w2_ref, gs_ref):
        dinv = dinv_ref[...]
        p = (oa_ref[...] + ob_ref[...] + xs_ref[...]) * dinv
        h1 = jnp.dot(p, w1_ref[...], preferred_element_type=jnp.float32)
        h1 = h1 + b1_ref[...]
        h1 = jnp.where(h1 > 0, h1, 0.01 * h1)
        g = jnp.dot(h1, w2_ref[...], preferred_element_type=jnp.float32)
        gs_ref[...] = g * dinv

    return pl.pallas_call(
        body,
        grid=(n // block,),
        in_specs=[
            pl.BlockSpec((block, f), lambda i: (i, 0)),
            pl.BlockSpec((block, f), lambda i: (i, 0)),
            pl.BlockSpec((block, f), lambda i: (i, 0)),
            pl.BlockSpec((block, 1), lambda i: (i, 0)),
            pl.BlockSpec(W1.shape, lambda i: (0, 0)),
            pl.BlockSpec((1, k1), lambda i: (0, 0)),
            pl.BlockSpec(W2p.shape, lambda i: (0, 0)),
        ],
        out_specs=pl.BlockSpec((block, f2), lambda i: (i, 0)),
        out_shape=jax.ShapeDtypeStruct((n, f2), jnp.float32),
    )(o1a, o1b, xs, dinv, W1, b1r, W2p)


def _tc_head(o2a, o2b, gs, dinv, b2p, W3p, b3r, block):
    n, f2 = gs.shape
    ncls = W3p.shape[1]

    def body(oa_ref, ob_ref, gs_ref, dinv_ref, b2_ref, w3_ref, b3_ref, out_ref):
        dinv = dinv_ref[...]
        pre = (oa_ref[...] + ob_ref[...] + gs_ref[...]) * dinv + b2_ref[...]
        h2 = jnp.where(pre > 0, pre, 0.01 * pre)
        logits = jnp.dot(h2, w3_ref[...], preferred_element_type=jnp.float32)
        logits = logits + b3_ref[...]
        m = jnp.max(logits, axis=1, keepdims=True)
        lse = jnp.log(jnp.sum(jnp.exp(logits - m), axis=1, keepdims=True)) + m
        out_ref[...] = logits - lse

    return pl.pallas_call(
        body,
        grid=(n // block,),
        in_specs=[
            pl.BlockSpec((block, f2), lambda i: (i, 0)),
            pl.BlockSpec((block, f2), lambda i: (i, 0)),
            pl.BlockSpec((block, f2), lambda i: (i, 0)),
            pl.BlockSpec((block, 1), lambda i: (i, 0)),
            pl.BlockSpec((1, f2), lambda i: (0, 0)),
            pl.BlockSpec(W3p.shape, lambda i: (0, 0)),
            pl.BlockSpec((1, ncls), lambda i: (0, 0)),
        ],
        out_specs=pl.BlockSpec((block, ncls), lambda i: (i, 0)),
        out_shape=jax.ShapeDtypeStruct((n, ncls), jnp.float32),
    )(o2a, o2b, gs, dinv, b2p, W3p, b3r)


def kernel(x, edge_index, W1, b1, W2, b2, W3, b3):
    n, f = x.shape
    e = edge_index.shape[1]
    ei = edge_index.astype(jnp.int32)
    src, dst = ei[0], ei[1]

    chunk = NW * K
    e_pad = ((e + chunk - 1) // chunk) * chunk
    nb = e_pad // chunk
    n_pad = ((n + 1 + NS * 8 - 1) // (NS * 8)) * (NS * 8)

    if e_pad != e:
        fill = jnp.full((e_pad - e,), n, jnp.int32)
        src = jnp.concatenate([src, fill])
        dst = jnp.concatenate([dst, fill])
    src2d = src.reshape(NW * nb, K)
    dst2d = dst.reshape(NW * nb, K)

    zeros_f = jnp.zeros((n_pad, f), jnp.float32)
    zeros_16 = jnp.zeros((n_pad, 16), jnp.float32)
    ones_k = jnp.ones((K, 16), jnp.float32)
    pad_rows = jnp.zeros((n_pad - n, f), jnp.float32)

    block = 1000 if n % 1000 == 0 else 8
    w1k = W1.shape[1]
    f2 = f
    W2p = jnp.pad(W2, ((0, 0), (0, f2 - W2.shape[1])))
    W3p = jnp.pad(W3, ((0, f2 - W3.shape[0]), (0, 0)))
    b2p = jnp.pad(b2, (0, f2 - b2.shape[0])).reshape(1, f2)
    b1r = b1.reshape(1, w1k)
    b3r = b3.reshape(1, W3.shape[1])

    deg_fn = _make_degree(n_pad, nb)
    scat_fn = _make_scatter_add(n_pad, f, nb)

    degp = deg_fn(dst2d, ones_k, zeros_16)
    dinv, xs = _tc_scale(degp[0], degp[1], x, block)

    xs_t = jnp.concatenate([xs, pad_rows])
    o1 = scat_fn(xs_t, src2d, dst2d, zeros_f)
    gs = _tc_layer1(o1[0], o1[1], xs, dinv, W1, b1r, W2p, block)

    gs_t = jnp.concatenate([gs, pad_rows])
    o2 = scat_fn(gs_t, src2d, dst2d, zeros_f)
    return _tc_head(o2[0], o2[1], gs, dinv, b2p, W3p, b3r, block)


# speedup vs baseline: 23.4678x; 23.4678x over previous
import functools

import jax
import jax.numpy as jnp
from jax import lax
from jax.experimental import pallas as pl
from jax.experimental.pallas import tpu as pltpu
from jax.experimental.pallas import tpu_sc as plsc

NC = 2
NS = 16
NW = NC * NS
K = 128

_MESH = plsc.VectorSubcoreMesh(core_axis_name="c", subcore_axis_name="s")


def _make_scatter_col(n_pad, fh, nb):
    rpt = n_pad // NS
    nbh = nb // 2

    @functools.partial(
        pl.kernel,
        out_type=jax.ShapeDtypeStruct((NC, n_pad, fh), jnp.float32),
        mesh=_MESH,
        compiler_params=pltpu.CompilerParams(needs_layout_passes=False,
                                             use_tc_tiling_on_sc=False),
        scratch_types=[
            pltpu.VMEM_SHARED((n_pad, fh), jnp.float32),
            pltpu.VMEM_SHARED((n_pad, fh), jnp.float32),
            pltpu.VMEM((nb // 2, K), jnp.int32),
            pltpu.VMEM((K,), jnp.int32),
            pltpu.VMEM((K,), jnp.int32),
            pltpu.VMEM((K,), jnp.int32),
            pltpu.VMEM((K,), jnp.int32),
            pltpu.VMEM((K, fh), jnp.float32),
            pltpu.VMEM((K, fh), jnp.float32),
            pltpu.SemaphoreType.DMA,
            pltpu.SemaphoreType.DMA,
        ],
    )
    def scatter_kernel(ta_hbm, tb_hbm, packed_hbm, zeros_hbm, out_hbm,
                       tbl, acc, pk, s0, s1, d0, d1, rows_a, rows_b,
                       sem_a, sem_b):
        c = lax.axis_index("c")
        s = lax.axis_index("s")
        r0 = s * rpt
        sl = pl.ds(r0, rpt)

        @pl.when(c == 0)
        def _():
            pltpu.sync_copy(ta_hbm.at[sl], tbl.at[sl])

        @pl.when(c == 1)
        def _():
            pltpu.sync_copy(tb_hbm.at[sl], tbl.at[sl])

        pltpu.sync_copy(zeros_hbm.at[sl], acc.at[sl])
        pltpu.sync_copy(packed_hbm.at[pl.ds(s * nb, nbh)], pk)
        plsc.subcore_barrier()

        def unpack(j, sbuf, dbuf):
            for l in range(K // 16):
                w = pk[j, pl.ds(16 * l, 16)]
                sbuf[pl.ds(16 * l, 16)] = lax.bitwise_and(w, 0xFFFF)
                dbuf[pl.ds(16 * l, 16)] = lax.shift_right_logical(w, 16)

        def start(sbuf, buf, sem):
            pltpu.async_copy(tbl.at[sbuf], buf, sem)

        def wait(buf, sem):
            pltpu.make_async_copy(tbl.at[s0], buf, sem).wait()

        def body(i, carry):
            j = 2 * i
            unpack(j + 1, s1, d1)
            start(s1, rows_b, sem_b)
            wait(rows_a, sem_a)
            pltpu.sync_copy(rows_a, acc.at[d0], add=True)

            @pl.when(j + 2 < nbh)
            def _():
                unpack(j + 2, s0, d0)
                start(s0, rows_a, sem_a)

            wait(rows_b, sem_b)
            pltpu.sync_copy(rows_b, acc.at[d1], add=True)
            return carry

        for half in range(2):
            if half == 1:
                pltpu.sync_copy(packed_hbm.at[pl.ds(s * nb + nbh, nbh)], pk)
            unpack(0, s0, d0)
            start(s0, rows_a, sem_a)
            lax.fori_loop(0, nbh // 2, body, 0)
        plsc.subcore_barrier()
        pltpu.sync_copy(acc.at[sl], out_hbm.at[c, sl])

    return scatter_kernel


def _make_degree(n_pad, nb):
    R = n_pad // 128

    @functools.partial(
        pl.kernel,
        out_type=jax.ShapeDtypeStruct((NC, R, 128), jnp.float32),
        mesh=_MESH,
        compiler_params=pltpu.CompilerParams(needs_layout_passes=False),
        scratch_types=[
            pltpu.VMEM_SHARED((R, 128), jnp.float32),
            pltpu.VMEM((nb, K), jnp.int32),
            pltpu.VMEM((R, 128), jnp.float32),
            pltpu.VMEM((R,), jnp.int32),
        ],
    )
    def deg_kernel(packed_hbm, zeros_hbm, out_hbm, acc, dst_v, hist, iota_v):
        c = lax.axis_index("c")
        s = lax.axis_index("s")
        wid = s * NC + c

        @pl.when(s == 0)
        def _():
            pltpu.sync_copy(zeros_hbm, acc)

        pltpu.sync_copy(zeros_hbm, hist)
        pltpu.sync_copy(packed_hbm.at[pl.ds(wid * nb, nb)], dst_v)
        for t in range(R // 16):
            iota_v[pl.ds(16 * t, 16)] = lax.iota(jnp.int32, 16) + 16 * t
        plsc.subcore_barrier()

        def body(j, carry):
            for l in range(K // 16):
                v = lax.shift_right_logical(dst_v[j, pl.ds(16 * l, 16)], 16)
                cnt, last = plsc.scan_count(v)
                row = lax.shift_right_logical(v, 7)
                col = lax.bitwise_and(v, 127)
                plsc.addupdate_scatter(
                    hist, [row, col], cnt.astype(jnp.float32), mask=last)
            return carry

        lax.fori_loop(0, nb, body, 0)
        pltpu.sync_copy(hist, acc.at[iota_v], add=True)
        plsc.subcore_barrier()

        @pl.when(s == 0)
        def _():
            pltpu.sync_copy(acc, out_hbm.at[c])

    return deg_kernel


def _tc_scale(dega, degb, x, block):
    n, f = x.shape
    fh = f // 2

    def body(da_ref, db_ref, x_ref, dinv_ref, lo_ref, hi_ref):
        d = da_ref[...] + db_ref[...] + 1.0
        dinv = 1.0 / jnp.sqrt(jnp.maximum(d, 1.0))
        dinv_ref[...] = dinv
        xs = x_ref[...] * dinv
        lo_ref[...] = xs[:, :fh]
        hi_ref[...] = xs[:, fh:]

    return pl.pallas_call(
        body,
        grid=(n // block,),
        in_specs=[
            pl.BlockSpec((block, 1), lambda i: (i, 0)),
            pl.BlockSpec((block, 1), lambda i: (i, 0)),
            pl.BlockSpec((block, f), lambda i: (i, 0)),
        ],
        out_specs=[
            pl.BlockSpec((block, 1), lambda i: (i, 0)),
            pl.BlockSpec((block, fh), lambda i: (i, 0)),
            pl.BlockSpec((block, fh), lambda i: (i, 0)),
        ],
        out_shape=[
            jax.ShapeDtypeStruct((n, 1), jnp.float32),
            jax.ShapeDtypeStruct((n, fh), jnp.float32),
            jax.ShapeDtypeStruct((n, fh), jnp.float32),
        ],
    )(dega, degb, x)


def _tc_layer1(o1, xs_lo, xs_hi, dinv, W1lo, W1hi, b1r, W2lo, W2hi, block):
    n, fh = xs_lo.shape
    k1 = W1lo.shape[1]

    def body(o_ref, xl_ref, xh_ref, dinv_ref, w1l_ref, w1h_ref, b1_ref,
             w2l_ref, w2h_ref, gl_ref, gh_ref):
        dinv = dinv_ref[...]
        a = o_ref[...]
        p_lo = (a[0] + xl_ref[...]) * dinv
        p_hi = (a[1] + xh_ref[...]) * dinv
        h1 = jnp.dot(p_lo, w1l_ref[...], preferred_element_type=jnp.float32)
        h1 = h1 + jnp.dot(p_hi, w1h_ref[...],
                          preferred_element_type=jnp.float32)
        h1 = h1 + b1_ref[...]
        h1 = jnp.where(h1 > 0, h1, 0.01 * h1)
        gl = jnp.dot(h1, w2l_ref[...], preferred_element_type=jnp.float32)
        gh = jnp.dot(h1, w2h_ref[...], preferred_element_type=jnp.float32)
        gl_ref[...] = gl * dinv
        gh_ref[...] = gh * dinv

    return pl.pallas_call(
        body,
        grid=(n // block,),
        in_specs=[
            pl.BlockSpec((NC, block, fh), lambda i: (0, i, 0)),
            pl.BlockSpec((block, fh), lambda i: (i, 0)),
            pl.BlockSpec((block, fh), lambda i: (i, 0)),
            pl.BlockSpec((block, 1), lambda i: (i, 0)),
            pl.BlockSpec(W1lo.shape, lambda i: (0, 0)),
            pl.BlockSpec(W1hi.shape, lambda i: (0, 0)),
            pl.BlockSpec((1, k1), lambda i: (0, 0)),
            pl.BlockSpec(W2lo.shape, lambda i: (0, 0)),
            pl.BlockSpec(W2hi.shape, lambda i: (0, 0)),
        ],
        out_specs=[
            pl.BlockSpec((block, fh), lambda i: (i, 0)),
            pl.BlockSpec((block, fh), lambda i: (i, 0)),
        ],
        out_shape=[
            jax.ShapeDtypeStruct((n, fh), jnp.float32),
            jax.ShapeDtypeStruct((n, fh), jnp.float32),
        ],
    )(o1, xs_lo, xs_hi, dinv, W1lo, W1hi, b1r, W2lo, W2hi)


def _tc_head(o2, gs_lo, gs_hi, dinv, b2lo, b2hi, W3lo, W3hi, b3r, block):
    n, fh = gs_lo.shape
    ncls = W3lo.shape[1]

    def body(o_ref, gl_ref, gh_ref, dinv_ref, b2l_ref, b2h_ref,
             w3l_ref, w3h_ref, b3_ref, out_ref):
        dinv = dinv_ref[...]
        a = o_ref[...]
        pre_lo = (a[0] + gl_ref[...]) * dinv + b2l_ref[...]
        pre_hi = (a[1] + gh_ref[...]) * dinv + b2h_ref[...]
        h2_lo = jnp.where(pre_lo > 0, pre_lo, 0.01 * pre_lo)
        h2_hi = jnp.where(pre_hi > 0, pre_hi, 0.01 * pre_hi)
        logits = jnp.dot(h2_lo, w3l_ref[...],
                         preferred_element_type=jnp.float32)
        logits = logits + jnp.dot(h2_hi, w3h_ref[...],
                                  preferred_element_type=jnp.float32)
        logits = logits + b3_ref[...]
        m = jnp.max(logits, axis=1, keepdims=True)
        lse = jnp.log(jnp.sum(jnp.exp(logits - m), axis=1, keepdims=True)) + m
        out_ref[...] = logits - lse

    return pl.pallas_call(
        body,
        grid=(n // block,),
        in_specs=[
            pl.BlockSpec((NC, block, fh), lambda i: (0, i, 0)),
            pl.BlockSpec((block, fh), lambda i: (i, 0)),
            pl.BlockSpec((block, fh), lambda i: (i, 0)),
            pl.BlockSpec((block, 1), lambda i: (i, 0)),
            pl.BlockSpec((1, fh), lambda i: (0, 0)),
            pl.BlockSpec((1, fh), lambda i: (0, 0)),
            pl.BlockSpec(W3lo.shape, lambda i: (0, 0)),
            pl.BlockSpec(W3hi.shape, lambda i: (0, 0)),
            pl.BlockSpec((1, ncls), lambda i: (0, 0)),
        ],
        out_specs=pl.BlockSpec((block, ncls), lambda i: (i, 0)),
        out_shape=jax.ShapeDtypeStruct((n, ncls), jnp.float32),
    )(o2, gs_lo, gs_hi, dinv, b2lo, b2hi, W3lo, W3hi, b3r)


def kernel(x, edge_index, W1, b1, W2, b2, W3, b3):
    n, f = x.shape
    fh = f // 2
    e = edge_index.shape[1]
    ei = edge_index.astype(jnp.int32)
    src, dst = ei[0], ei[1]

    chunk = NS * K * 16
    e_pad = ((e + chunk - 1) // chunk) * chunk
    nb = e_pad // (NS * K)
    n_pad = ((n + 1 + 2047) // 2048) * 2048

    if e_pad != e:
        fill = jnp.full((e_pad - e,), n, jnp.int32)
        src = jnp.concatenate([src, fill])
        dst = jnp.concatenate([dst, fill])
    packed = (src | (dst << 16)).reshape(NS * nb, K)

    zeros_h = jnp.zeros((n_pad, fh), jnp.float32)
    zeros_r = jnp.zeros((n_pad // 128, 128), jnp.float32)
    x_p = jnp.concatenate([x, jnp.zeros((n_pad - n, f), jnp.float32)])

    block = 640 if n_pad % 640 == 0 else 128
    w1k = W1.shape[1]
    W2p = jnp.pad(W2, ((0, 0), (0, f - W2.shape[1])))
    W3p = jnp.pad(W3, ((0, f - W3.shape[0]), (0, 0)))
    b2p = jnp.pad(b2, (0, f - b2.shape[0]))
    W1lo, W1hi = W1[:fh], W1[fh:]
    W2lo, W2hi = W2p[:, :fh], W2p[:, fh:]
    W3lo, W3hi = W3p[:fh], W3p[fh:]
    b2lo, b2hi = b2p[:fh].reshape(1, fh), b2p[fh:].reshape(1, fh)
    b1r = b1.reshape(1, w1k)
    b3r = b3.reshape(1, W3.shape[1])

    deg_fn = _make_degree(n_pad, nb // 2)
    scat_fn = _make_scatter_col(n_pad, fh, nb)

    degp = deg_fn(packed, zeros_r)
    da = degp[0].reshape(n_pad, 1)
    db = degp[1].reshape(n_pad, 1)
    dinv, xs_lo, xs_hi = _tc_scale(da, db, x_p, block)

    o1 = scat_fn(xs_lo, xs_hi, packed, zeros_h)
    gs_lo, gs_hi = _tc_layer1(o1, xs_lo, xs_hi, dinv,
                              W1lo, W1hi, b1r, W2lo, W2hi, block)

    o2 = scat_fn(gs_lo, gs_hi, packed, zeros_h)
    out = _tc_head(o2, gs_lo, gs_hi, dinv, b2lo, b2hi, W3lo, W3hi, b3r, block)
    return out[:n]


# speedup vs baseline: 23.7029x; 1.0100x over previous
import functools

import jax
import jax.numpy as jnp
from jax import lax
from jax.experimental import pallas as pl
from jax.experimental.pallas import tpu as pltpu
from jax.experimental.pallas import tpu_sc as plsc

NC = 2
NS = 16
NW = NC * NS
K = 128

_MESH = plsc.VectorSubcoreMesh(core_axis_name="c", subcore_axis_name="s")


def _make_scatter_col(n_pad, fh, nb):
    rpt = n_pad // NS

    @functools.partial(
        pl.kernel,
        out_type=jax.ShapeDtypeStruct((NC, n_pad, fh), jnp.float32),
        mesh=_MESH,
        compiler_params=pltpu.CompilerParams(needs_layout_passes=False,
                                             use_tc_tiling_on_sc=False),
        scratch_types=[
            pltpu.VMEM_SHARED((n_pad, fh), jnp.float32),
            pltpu.VMEM_SHARED((n_pad, fh), jnp.float32),
            pltpu.VMEM((nb, K), jnp.int32),
            pltpu.VMEM((K,), jnp.int32),
            pltpu.VMEM((K,), jnp.int32),
            pltpu.VMEM((K,), jnp.int32),
            pltpu.VMEM((K,), jnp.int32),
            pltpu.VMEM((K, fh), jnp.float32),
            pltpu.VMEM((K, fh), jnp.float32),
            pltpu.SemaphoreType.DMA,
            pltpu.SemaphoreType.DMA,
        ],
    )
    def scatter_kernel(ta_hbm, tb_hbm, packed_hbm, zeros_hbm, out_hbm,
                       tbl, acc, pk, s0, s1, d0, d1, rows_a, rows_b,
                       sem_a, sem_b):
        c = lax.axis_index("c")
        s = lax.axis_index("s")
        r0 = s * rpt
        sl = pl.ds(r0, rpt)

        @pl.when(c == 0)
        def _():
            pltpu.sync_copy(ta_hbm.at[sl], tbl.at[sl])

        @pl.when(c == 1)
        def _():
            pltpu.sync_copy(tb_hbm.at[sl], tbl.at[sl])

        pltpu.sync_copy(zeros_hbm.at[sl], acc.at[sl])
        pltpu.sync_copy(packed_hbm.at[pl.ds(s * nb, nb)], pk)
        plsc.subcore_barrier()

        def unpack(j, sbuf, dbuf):
            for l in range(K // 16):
                w = pk[j, pl.ds(16 * l, 16)]
                sbuf[pl.ds(16 * l, 16)] = lax.bitwise_and(w, 0xFFFF)
                dbuf[pl.ds(16 * l, 16)] = lax.shift_right_logical(w, 16)

        def start(sbuf, buf, sem):
            pltpu.async_copy(tbl.at[sbuf], buf, sem)

        def wait(buf, sem):
            pltpu.make_async_copy(tbl.at[s0], buf, sem).wait()

        def body(i, carry):
            j = 2 * i
            unpack(j + 1, s1, d1)
            start(s1, rows_b, sem_b)
            wait(rows_a, sem_a)
            pltpu.sync_copy(rows_a, acc.at[d0], add=True)

            @pl.when(j + 2 < nb)
            def _():
                unpack(j + 2, s0, d0)
                start(s0, rows_a, sem_a)

            wait(rows_b, sem_b)
            pltpu.sync_copy(rows_b, acc.at[d1], add=True)
            return carry

        unpack(0, s0, d0)
        start(s0, rows_a, sem_a)
        lax.fori_loop(0, nb // 2, body, 0)
        plsc.subcore_barrier()
        pltpu.sync_copy(acc.at[sl], out_hbm.at[c, sl])

    return scatter_kernel


def _make_degree(n_pad, nb):
    R = n_pad // 128

    @functools.partial(
        pl.kernel,
        out_type=jax.ShapeDtypeStruct((NC, R, 128), jnp.float32),
        mesh=_MESH,
        compiler_params=pltpu.CompilerParams(needs_layout_passes=False),
        scratch_types=[
            pltpu.VMEM_SHARED((R, 128), jnp.float32),
            pltpu.VMEM((nb, K), jnp.int32),
            pltpu.VMEM((R, 128), jnp.float32),
            pltpu.VMEM((R,), jnp.int32),
        ],
    )
    def deg_kernel(packed_hbm, zeros_hbm, out_hbm, acc, dst_v, hist, iota_v):
        c = lax.axis_index("c")
        s = lax.axis_index("s")
        wid = s * NC + c

        @pl.when(s == 0)
        def _():
            pltpu.sync_copy(zeros_hbm, acc)

        pltpu.sync_copy(zeros_hbm, hist)
        pltpu.sync_copy(packed_hbm.at[pl.ds(wid * nb, nb)], dst_v)
        for t in range(R // 16):
            iota_v[pl.ds(16 * t, 16)] = lax.iota(jnp.int32, 16) + 16 * t
        plsc.subcore_barrier()

        def body(j, carry):
            for l in range(K // 16):
                v = lax.shift_right_logical(dst_v[j, pl.ds(16 * l, 16)], 16)
                cnt, last = plsc.scan_count(v)
                row = lax.shift_right_logical(v, 7)
                col = lax.bitwise_and(v, 127)
                plsc.addupdate_scatter(
                    hist, [row, col], cnt.astype(jnp.float32), mask=last)
            return carry

        lax.fori_loop(0, nb, body, 0)
        pltpu.sync_copy(hist, acc.at[iota_v], add=True)
        plsc.subcore_barrier()

        @pl.when(s == 0)
        def _():
            pltpu.sync_copy(acc, out_hbm.at[c])

    return deg_kernel


def _tc_scale(dega, degb, x, block):
    n, f = x.shape
    fh = f // 2

    def body(da_ref, db_ref, x_ref, dinv_ref, lo_ref, hi_ref):
        d = da_ref[...] + db_ref[...] + 1.0
        dinv = 1.0 / jnp.sqrt(jnp.maximum(d, 1.0))
        dinv_ref[...] = dinv
        xs = x_ref[...] * dinv
        lo_ref[...] = xs[:, :fh]
        hi_ref[...] = xs[:, fh:]

    return pl.pallas_call(
        body,
        grid=(n // block,),
        in_specs=[
            pl.BlockSpec((block, 1), lambda i: (i, 0)),
            pl.BlockSpec((block, 1), lambda i: (i, 0)),
            pl.BlockSpec((block, f), lambda i: (i, 0)),
        ],
        out_specs=[
            pl.BlockSpec((block, 1), lambda i: (i, 0)),
            pl.BlockSpec((block, fh), lambda i: (i, 0)),
            pl.BlockSpec((block, fh), lambda i: (i, 0)),
        ],
        out_shape=[
            jax.ShapeDtypeStruct((n, 1), jnp.float32),
            jax.ShapeDtypeStruct((n, fh), jnp.float32),
            jax.ShapeDtypeStruct((n, fh), jnp.float32),
        ],
    )(dega, degb, x)


def _tc_layer1(o1, xs_lo, xs_hi, dinv, W1lo, W1hi, b1r, W2lo, W2hi, block):
    n, fh = xs_lo.shape
    k1 = W1lo.shape[1]

    def body(o_ref, xl_ref, xh_ref, dinv_ref, w1l_ref, w1h_ref, b1_ref,
             w2l_ref, w2h_ref, gl_ref, gh_ref):
        dinv = dinv_ref[...]
        a = o_ref[...]
        p_lo = (a[0] + xl_ref[...]) * dinv
        p_hi = (a[1] + xh_ref[...]) * dinv
        h1 = jnp.dot(p_lo, w1l_ref[...], preferred_element_type=jnp.float32)
        h1 = h1 + jnp.dot(p_hi, w1h_ref[...],
                          preferred_element_type=jnp.float32)
        h1 = h1 + b1_ref[...]
        h1 = jnp.where(h1 > 0, h1, 0.01 * h1)
        gl = jnp.dot(h1, w2l_ref[...], preferred_element_type=jnp.float32)
        gh = jnp.dot(h1, w2h_ref[...], preferred_element_type=jnp.float32)
        gl_ref[...] = gl * dinv
        gh_ref[...] = gh * dinv

    return pl.pallas_call(
        body,
        grid=(n // block,),
        in_specs=[
            pl.BlockSpec((NC, block, fh), lambda i: (0, i, 0)),
            pl.BlockSpec((block, fh), lambda i: (i, 0)),
            pl.BlockSpec((block, fh), lambda i: (i, 0)),
            pl.BlockSpec((block, 1), lambda i: (i, 0)),
            pl.BlockSpec(W1lo.shape, lambda i: (0, 0)),
            pl.BlockSpec(W1hi.shape, lambda i: (0, 0)),
            pl.BlockSpec((1, k1), lambda i: (0, 0)),
            pl.BlockSpec(W2lo.shape, lambda i: (0, 0)),
            pl.BlockSpec(W2hi.shape, lambda i: (0, 0)),
        ],
        out_specs=[
            pl.BlockSpec((block, fh), lambda i: (i, 0)),
            pl.BlockSpec((block, fh), lambda i: (i, 0)),
        ],
        out_shape=[
            jax.ShapeDtypeStruct((n, fh), jnp.float32),
            jax.ShapeDtypeStruct((n, fh), jnp.float32),
        ],
    )(o1, xs_lo, xs_hi, dinv, W1lo, W1hi, b1r, W2lo, W2hi)


def _tc_head(o2, gs_lo, gs_hi, dinv, b2lo, b2hi, W3lo, W3hi, b3r, block):
    n, fh = gs_lo.shape
    ncls = W3lo.shape[1]

    def body(o_ref, gl_ref, gh_ref, dinv_ref, b2l_ref, b2h_ref,
             w3l_ref, w3h_ref, b3_ref, out_ref):
        dinv = dinv_ref[...]
        a = o_ref[...]
        pre_lo = (a[0] + gl_ref[...]) * dinv + b2l_ref[...]
        pre_hi = (a[1] + gh_ref[...]) * dinv + b2h_ref[...]
        h2_lo = jnp.where(pre_lo > 0, pre_lo, 0.01 * pre_lo)
        h2_hi = jnp.where(pre_hi > 0, pre_hi, 0.01 * pre_hi)
        logits = jnp.dot(h2_lo, w3l_ref[...],
                         preferred_element_type=jnp.float32)
        logits = logits + jnp.dot(h2_hi, w3h_ref[...],
                                  preferred_element_type=jnp.float32)
        logits = logits + b3_ref[...]
        m = jnp.max(logits, axis=1, keepdims=True)
        lse = jnp.log(jnp.sum(jnp.exp(logits - m), axis=1, keepdims=True)) + m
        out_ref[...] = logits - lse

    return pl.pallas_call(
        body,
        grid=(n // block,),
        in_specs=[
            pl.BlockSpec((NC, block, fh), lambda i: (0, i, 0)),
            pl.BlockSpec((block, fh), lambda i: (i, 0)),
            pl.BlockSpec((block, fh), lambda i: (i, 0)),
            pl.BlockSpec((block, 1), lambda i: (i, 0)),
            pl.BlockSpec((1, fh), lambda i: (0, 0)),
            pl.BlockSpec((1, fh), lambda i: (0, 0)),
            pl.BlockSpec(W3lo.shape, lambda i: (0, 0)),
            pl.BlockSpec(W3hi.shape, lambda i: (0, 0)),
            pl.BlockSpec((1, ncls), lambda i: (0, 0)),
        ],
        out_specs=pl.BlockSpec((block, ncls), lambda i: (i, 0)),
        out_shape=jax.ShapeDtypeStruct((n, ncls), jnp.float32),
    )(o2, gs_lo, gs_hi, dinv, b2lo, b2hi, W3lo, W3hi, b3r)


def kernel(x, edge_index, W1, b1, W2, b2, W3, b3):
    n, f = x.shape
    fh = f // 2
    e = edge_index.shape[1]
    ei = edge_index.astype(jnp.int32)
    src, dst = ei[0], ei[1]

    chunk = NS * K * 16
    e_pad = ((e + chunk - 1) // chunk) * chunk
    nb = e_pad // (NS * K)
    n_pad = ((n + 1 + 2047) // 2048) * 2048

    if e_pad != e:
        fill = jnp.full((e_pad - e,), n, jnp.int32)
        src = jnp.concatenate([src, fill])
        dst = jnp.concatenate([dst, fill])
    packed = (src | (dst << 16)).reshape(NS * nb, K)

    zeros_h = jnp.zeros((n_pad, fh), jnp.float32)
    zeros_r = jnp.zeros((n_pad // 128, 128), jnp.float32)
    x_p = jnp.concatenate([x, jnp.zeros((n_pad - n, f), jnp.float32)])

    block = 640 if n_pad % 640 == 0 else 128
    w1k = W1.shape[1]
    W2p = jnp.pad(W2, ((0, 0), (0, f - W2.shape[1])))
    W3p = jnp.pad(W3, ((0, f - W3.shape[0]), (0, 0)))
    b2p = jnp.pad(b2, (0, f - b2.shape[0]))
    W1lo, W1hi = W1[:fh], W1[fh:]
    W2lo, W2hi = W2p[:, :fh], W2p[:, fh:]
    W3lo, W3hi = W3p[:fh], W3p[fh:]
    b2lo, b2hi = b2p[:fh].reshape(1, fh), b2p[fh:].reshape(1, fh)
    b1r = b1.reshape(1, w1k)
    b3r = b3.reshape(1, W3.shape[1])

    deg_fn = _make_degree(n_pad, nb // 2)
    scat_fn = _make_scatter_col(n_pad, fh, nb)

    degp = deg_fn(packed, zeros_r)
    da = degp[0].reshape(n_pad, 1)
    db = degp[1].reshape(n_pad, 1)
    dinv, xs_lo, xs_hi = _tc_scale(da, db, x_p, block)

    o1 = scat_fn(xs_lo, xs_hi, packed, zeros_h)
    gs_lo, gs_hi = _tc_layer1(o1, xs_lo, xs_hi, dinv,
                              W1lo, W1hi, b1r, W2lo, W2hi, block)

    o2 = scat_fn(gs_lo, gs_hi, packed, zeros_h)
    out = _tc_head(o2, gs_lo, gs_hi, dinv, b2lo, b2hi, W3lo, W3hi, b3r, block)
    return out[:n]
